# Initial kernel scaffold; baseline (speedup 1.0000x reference)
import jax
import jax.numpy as jnp
from jax.experimental import pallas as pl


def kernel(x, edge_index, W1, b1, W2, b2):
    raise NotImplementedError("write your pallas kernel here")


# speedup vs baseline: 36.6754x; 36.6754x over previous
import functools

import jax
import jax.numpy as jnp
from jax import lax
from jax.experimental import pallas as pl
from jax.experimental.pallas import tpu as pltpu
from jax.experimental.pallas import tpu_sc as plsc

N_NODES = 100000
N_EDGES = 6400000

ROW = 128
BR = 16
NROWS = N_EDGES // ROW
NBLK = NROWS // BR
NCORE = 2
NSUB = 16
NW = NCORE * NSUB
ITERS = -(-NBLK // NW)
NPAD = 100352
ZR = NPAD // NSUB
LANES = NPAD // 128


def _mesh():
    return plsc.VectorSubcoreMesh(core_axis_name="c", subcore_axis_name="s")


_SC_PARAMS = pltpu.CompilerParams(use_tc_tiling_on_sc=False,
                                  needs_layout_passes=False)


@functools.partial(
    pl.kernel,
    out_type=jax.ShapeDtypeStruct((NCORE * NPAD,), jnp.float32),
    mesh=_mesh(),
    compiler_params=_SC_PARAMS,
    scratch_types=[
        pltpu.VMEM((BR, ROW), jnp.int32),
        pltpu.VMEM((ROW,), jnp.float32),
        pltpu.VMEM_SHARED((NPAD,), jnp.float32),
        pltpu.SemaphoreType.DMA,
    ],
)
def _deg_kernel(dst_hbm, zeros_hbm, ones_hbm, out_hbm, didx, ones_v, acc, sem):
    c = lax.axis_index("c")
    s = lax.axis_index("s")
    wid = s * NCORE + c
    pltpu.sync_copy(ones_hbm, ones_v)
    pltpu.sync_copy(zeros_hbm, acc.at[pl.ds(s * ZR, ZR)])
    plsc.subcore_barrier()

    def body(j, carry):
        b = wid + j * NW

        @pl.when(b < NBLK)
        def _():
            pltpu.sync_copy(dst_hbm.at[pl.ds(b * BR, BR)], didx)

            def rbody(r, rcarry):
                d = pltpu.async_copy(ones_v, acc.at[didx.at[r]], sem,
                                     add=True)
                d.wait()
                return rcarry

            lax.fori_loop(0, BR, rbody, 0)

        return carry

    lax.fori_loop(0, ITERS, body, 0)
    plsc.subcore_barrier()
    pltpu.sync_copy(acc.at[pl.ds(s * ZR, ZR)],
                    out_hbm.at[pl.ds(c * NPAD + s * ZR, ZR)])


def _make_scatter_kernel(feat):
    @functools.partial(
        pl.kernel,
        out_type=jax.ShapeDtypeStruct((NCORE, feat, NPAD), jnp.float32),
        mesh=_mesh(),
        compiler_params=_SC_PARAMS,
        scratch_types=[
            pltpu.VMEM((BR, ROW), jnp.int32),
            pltpu.VMEM((BR, ROW), jnp.int32),
            pltpu.VMEM((feat, ROW), jnp.float32),
        ] + [
            pltpu.VMEM_SHARED((NPAD,), jnp.float32)
            for _ in range(feat)
        ] + [
            pltpu.SemaphoreType.DMA,
            pltpu.SemaphoreType.DMA,
        ],
    )
    def _k(src_hbm, dst_hbm, *cols_and_rest):
        cols_hbm = cols_and_rest[:feat]
        zeros_hbm = cols_and_rest[feat]
        out_hbm = cols_and_rest[feat + 1]
        sidx = cols_and_rest[feat + 2]
        didx = cols_and_rest[feat + 3]
        vals = cols_and_rest[feat + 4]
        accs = cols_and_rest[feat + 5:feat + 5 + feat]
        gsem = cols_and_rest[feat + 5 + feat]
        ssem = cols_and_rest[feat + 6 + feat]

        c = lax.axis_index("c")
        s = lax.axis_index("s")
        wid = s * NCORE + c
        for k in range(feat):
            pltpu.sync_copy(zeros_hbm, accs[k].at[pl.ds(s * ZR, ZR)])
        plsc.subcore_barrier()

        def body(j, carry):
            b = wid + j * NW

            @pl.when(b < NBLK)
            def _():
                pltpu.sync_copy(src_hbm.at[pl.ds(b * BR, BR)], sidx)
                pltpu.sync_copy(dst_hbm.at[pl.ds(b * BR, BR)], didx)

                def rbody(r, rcarry):
                    gd = [
                        pltpu.async_copy(cols_hbm[k].at[sidx.at[r]],
                                         vals.at[k], gsem)
                        for k in range(feat)
                    ]
                    for d in gd:
                        d.wait()
                    sd = [
                        pltpu.async_copy(vals.at[k], accs[k].at[didx.at[r]],
                                         ssem, add=True)
                        for k in range(feat)
                    ]
                    for d in sd:
                        d.wait()
                    return rcarry

                lax.fori_loop(0, BR, rbody, 0)

            return carry

        lax.fori_loop(0, ITERS, body, 0)
        plsc.subcore_barrier()
        for k in range(feat):
            pltpu.sync_copy(accs[k].at[pl.ds(s * ZR, ZR)],
                            out_hbm.at[c, k, pl.ds(s * ZR, ZR)])

    return _k


_scatter4 = _make_scatter_kernel(4)
_scatter2 = _make_scatter_kernel(2)


def _tc1_body(degp_ref, xc_ref, w1_ref, dis_ref, g1_ref):
    deg = degp_ref[0] + degp_ref[1] + 1.0
    dis = lax.rsqrt(deg)
    dis_ref[...] = dis
    xs = [xc_ref[k] for k in range(3)]
    for j in range(4):
        h = xs[0] * w1_ref[0, j] + xs[1] * w1_ref[1, j] + xs[2] * w1_ref[2, j]
        g1_ref[j] = h * dis


def _tc2_body(t1p_ref, g1_ref, dis_ref, b1_ref, w2_ref, g2_ref):
    dis = dis_ref[...]
    hs = []
    for k in range(4):
        t = (t1p_ref[0, k] + t1p_ref[1, k] + g1_ref[k]) * dis + b1_ref[k]
        hs.append(jnp.maximum(t, 0.0))
    for j in range(2):
        h = (hs[0] * w2_ref[0, j] + hs[1] * w2_ref[1, j]
             + hs[2] * w2_ref[2, j] + hs[3] * w2_ref[3, j])
        g2_ref[j] = h * dis


def _tc3_body(t2p_ref, g2_ref, dis_ref, b2_ref, out_ref):
    dis = dis_ref[...]
    h0 = (t2p_ref[0, 0] + t2p_ref[1, 0] + g2_ref[0]) * dis + b2_ref[0]
    h1 = (t2p_ref[0, 1] + t2p_ref[1, 1] + g2_ref[1]) * dis + b2_ref[1]
    m = jnp.maximum(h0, h1)
    lse = m + jnp.log(jnp.exp(h0 - m) + jnp.exp(h1 - m))
    out_ref[0] = h0 - lse
    out_ref[1] = h1 - lse


def kernel(x, edge_index, W1, b1, W2, b2):
    f32 = jnp.float32
    ei = edge_index.astype(jnp.int32)
    src = ei[0].reshape(NROWS, ROW)
    dst = ei[1].reshape(NROWS, ROW)

    xcols = jnp.pad(x, ((0, NPAD - N_NODES), (0, 0))).T.reshape(3, LANES, 128)

    zeros1 = jnp.zeros((ZR,), f32)
    ones = jnp.ones((ROW,), f32)

    vmem = pl.BlockSpec(memory_space=pltpu.VMEM)
    smem = pl.BlockSpec(memory_space=pltpu.SMEM)

    degp = _deg_kernel(dst, zeros1, ones).reshape(NCORE, LANES, 128)

    dis, g1 = pl.pallas_call(
        _tc1_body,
        out_shape=[
            jax.ShapeDtypeStruct((LANES, 128), f32),
            jax.ShapeDtypeStruct((4, LANES, 128), f32),
        ],
        in_specs=[vmem, vmem, smem],
        out_specs=[vmem, vmem],
    )(degp, xcols, W1)

    g1f = g1.reshape(4, NPAD)
    t1p = _scatter4(src, dst, g1f[0], g1f[1], g1f[2], g1f[3], zeros1)

    g2 = pl.pallas_call(
        _tc2_body,
        out_shape=jax.ShapeDtypeStruct((2, LANES, 128), f32),
        in_specs=[vmem, vmem, vmem, smem, smem],
        out_specs=vmem,
    )(t1p.reshape(NCORE, 4, LANES, 128), g1, dis, b1, W2)

    g2f = g2.reshape(2, NPAD)
    t2p = _scatter2(src, dst, g2f[0], g2f[1], zeros1)

    outc = pl.pallas_call(
        _tc3_body,
        out_shape=jax.ShapeDtypeStruct((2, LANES, 128), f32),
        in_specs=[vmem, vmem, vmem, smem],
        out_specs=vmem,
    )(t2p.reshape(NCORE, 2, LANES, 128), g2, dis, b2)

    return outc.reshape(2, NPAD)[:, :N_NODES].T


# speedup vs baseline: 59.6812x; 1.6273x over previous
import functools

import jax
import jax.numpy as jnp
from jax import lax
from jax.experimental import pallas as pl
from jax.experimental.pallas import tpu as pltpu
from jax.experimental.pallas import tpu_sc as plsc

N_NODES = 100000
N_EDGES = 6400000

ROW = 128
BR = 8
NROWS = N_EDGES // ROW
NBLK = NROWS // BR
NCORE = 2
NSUB = 16
NW = NCORE * NSUB
ITERS = -(-NBLK // NW)
NPAD = 100352
ZR = NPAD // NSUB
LANES = NPAD // 128


def _mesh():
    return plsc.VectorSubcoreMesh(core_axis_name="c", subcore_axis_name="s")


_SC_PARAMS = pltpu.CompilerParams(use_tc_tiling_on_sc=False,
                                  needs_layout_passes=False)


@functools.partial(
    pl.kernel,
    out_type=jax.ShapeDtypeStruct((NCORE * NPAD,), jnp.float32),
    mesh=_mesh(),
    compiler_params=_SC_PARAMS,
    scratch_types=[
        pltpu.VMEM((BR, ROW), jnp.int32),
        pltpu.VMEM((ROW,), jnp.float32),
        pltpu.VMEM_SHARED((NPAD,), jnp.float32),
        pltpu.SemaphoreType.DMA,
    ],
)
def _deg_kernel(dst_hbm, zeros_hbm, ones_hbm, out_hbm, didx, ones_v, acc, sem):
    c = lax.axis_index("c")
    s = lax.axis_index("s")
    wid = s * NCORE + c
    pltpu.sync_copy(ones_hbm, ones_v)
    pltpu.sync_copy(zeros_hbm, acc.at[pl.ds(s * ZR, ZR)])
    plsc.subcore_barrier()

    def body(j, carry):
        b = wid + j * NW

        @pl.when(b < NBLK)
        def _():
            pltpu.sync_copy(dst_hbm.at[pl.ds(b * BR, BR)], didx)
            sd = [
                pltpu.async_copy(ones_v, acc.at[didx.at[r]], sem, add=True)
                for r in range(BR)
            ]
            for d in sd:
                d.wait()

        return carry

    lax.fori_loop(0, ITERS, body, 0)
    plsc.subcore_barrier()
    pltpu.sync_copy(acc.at[pl.ds(s * ZR, ZR)],
                    out_hbm.at[pl.ds(c * NPAD + s * ZR, ZR)])


def _make_scatter_kernel(feat):
    @functools.partial(
        pl.kernel,
        out_type=jax.ShapeDtypeStruct((NCORE, feat, NPAD), jnp.float32),
        mesh=_mesh(),
        compiler_params=_SC_PARAMS,
        scratch_types=[
            pltpu.VMEM((BR, ROW), jnp.int32),
            pltpu.VMEM((BR, ROW), jnp.int32),
            pltpu.VMEM((BR * feat, ROW), jnp.float32),
        ] + [
            pltpu.VMEM_SHARED((NPAD,), jnp.float32)
            for _ in range(feat)
        ] + [
            pltpu.SemaphoreType.DMA,
            pltpu.SemaphoreType.DMA,
        ],
    )
    def _k(src_hbm, dst_hbm, *cols_and_rest):
        cols_hbm = cols_and_rest[:feat]
        zeros_hbm = cols_and_rest[feat]
        out_hbm = cols_and_rest[feat + 1]
        sidx = cols_and_rest[feat + 2]
        didx = cols_and_rest[feat + 3]
        vals = cols_and_rest[feat + 4]
        accs = cols_and_rest[feat + 5:feat + 5 + feat]
        gsem = cols_and_rest[feat + 5 + feat]
        ssem = cols_and_rest[feat + 6 + feat]

        c = lax.axis_index("c")
        s = lax.axis_index("s")
        wid = s * NCORE + c
        for k in range(feat):
            pltpu.sync_copy(zeros_hbm, accs[k].at[pl.ds(s * ZR, ZR)])
        plsc.subcore_barrier()

        def body(j, carry):
            b = wid + j * NW

            @pl.when(b < NBLK)
            def _():
                pltpu.sync_copy(src_hbm.at[pl.ds(b * BR, BR)], sidx)
                pltpu.sync_copy(dst_hbm.at[pl.ds(b * BR, BR)], didx)
                gd = [
                    pltpu.async_copy(cols_hbm[k].at[sidx.at[r]],
                                     vals.at[r * feat + k], gsem)
                    for r in range(BR)
                    for k in range(feat)
                ]
                for d in gd:
                    d.wait()
                sd = [
                    pltpu.async_copy(vals.at[r * feat + k],
                                     accs[k].at[didx.at[r]], ssem, add=True)
                    for r in range(BR)
                    for k in range(feat)
                ]
                for d in sd:
                    d.wait()

            return carry

        lax.fori_loop(0, ITERS, body, 0)
        plsc.subcore_barrier()
        for k in range(feat):
            pltpu.sync_copy(accs[k].at[pl.ds(s * ZR, ZR)],
                            out_hbm.at[c, k, pl.ds(s * ZR, ZR)])

    return _k


_scatter4 = _make_scatter_kernel(4)
_scatter2 = _make_scatter_kernel(2)


def _tc1_body(degp_ref, xc_ref, w1_ref, dis_ref, g1_ref):
    deg = degp_ref[0] + degp_ref[1] + 1.0
    dis = lax.rsqrt(deg)
    dis_ref[...] = dis
    xs = [xc_ref[k] for k in range(3)]
    for j in range(4):
        h = xs[0] * w1_ref[0, j] + xs[1] * w1_ref[1, j] + xs[2] * w1_ref[2, j]
        g1_ref[j] = h * dis


def _tc2_body(t1p_ref, g1_ref, dis_ref, b1_ref, w2_ref, g2_ref):
    dis = dis_ref[...]
    hs = []
    for k in range(4):
        t = (t1p_ref[0, k] + t1p_ref[1, k] + g1_ref[k]) * dis + b1_ref[k]
        hs.append(jnp.maximum(t, 0.0))
    for j in range(2):
        h = (hs[0] * w2_ref[0, j] + hs[1] * w2_ref[1, j]
             + hs[2] * w2_ref[2, j] + hs[3] * w2_ref[3, j])
        g2_ref[j] = h * dis


def _tc3_body(t2p_ref, g2_ref, dis_ref, b2_ref, out_ref):
    dis = dis_ref[...]
    h0 = (t2p_ref[0, 0] + t2p_ref[1, 0] + g2_ref[0]) * dis + b2_ref[0]
    h1 = (t2p_ref[0, 1] + t2p_ref[1, 1] + g2_ref[1]) * dis + b2_ref[1]
    m = jnp.maximum(h0, h1)
    lse = m + jnp.log(jnp.exp(h0 - m) + jnp.exp(h1 - m))
    out_ref[0] = h0 - lse
    out_ref[1] = h1 - lse


def kernel(x, edge_index, W1, b1, W2, b2):
    f32 = jnp.float32
    ei = edge_index.astype(jnp.int32)
    src = ei[0].reshape(NROWS, ROW)
    dst = ei[1].reshape(NROWS, ROW)

    xcols = jnp.pad(x, ((0, NPAD - N_NODES), (0, 0))).T.reshape(3, LANES, 128)

    zeros1 = jnp.zeros((ZR,), f32)
    ones = jnp.ones((ROW,), f32)

    vmem = pl.BlockSpec(memory_space=pltpu.VMEM)
    smem = pl.BlockSpec(memory_space=pltpu.SMEM)

    degp = _deg_kernel(dst, zeros1, ones).reshape(NCORE, LANES, 128)

    dis, g1 = pl.pallas_call(
        _tc1_body,
        out_shape=[
            jax.ShapeDtypeStruct((LANES, 128), f32),
            jax.ShapeDtypeStruct((4, LANES, 128), f32),
        ],
        in_specs=[vmem, vmem, smem],
        out_specs=[vmem, vmem],
    )(degp, xcols, W1)

    g1f = g1.reshape(4, NPAD)
    t1p = _scatter4(src, dst, g1f[0], g1f[1], g1f[2], g1f[3], zeros1)

    g2 = pl.pallas_call(
        _tc2_body,
        out_shape=jax.ShapeDtypeStruct((2, LANES, 128), f32),
        in_specs=[vmem, vmem, vmem, smem, smem],
        out_specs=vmem,
    )(t1p.reshape(NCORE, 4, LANES, 128), g1, dis, b1, W2)

    g2f = g2.reshape(2, NPAD)
    t2p = _scatter2(src, dst, g2f[0], g2f[1], zeros1)

    outc = pl.pallas_call(
        _tc3_body,
        out_shape=jax.ShapeDtypeStruct((2, LANES, 128), f32),
        in_specs=[vmem, vmem, vmem, smem],
        out_specs=vmem,
    )(t2p.reshape(NCORE, 2, LANES, 128), g2, dis, b2)

    return outc.reshape(2, NPAD)[:, :N_NODES].T


# speedup vs baseline: 83.4400x; 1.3981x over previous
import functools

import jax
import jax.numpy as jnp
from jax import lax
from jax.experimental import pallas as pl
from jax.experimental.pallas import tpu as pltpu
from jax.experimental.pallas import tpu_sc as plsc

N_NODES = 100000
N_EDGES = 6400000

ROW = 128
BR = 8
NROWS = N_EDGES // ROW
NBLK = NROWS // BR
NCORE = 2
NSUB = 16
NW = NCORE * NSUB
ITERS = -(-NBLK // NW)
NPAD = 100352
ZR = NPAD // NSUB
LANES = NPAD // 128


def _mesh():
    return plsc.VectorSubcoreMesh(core_axis_name="c", subcore_axis_name="s")


_SC_PARAMS = pltpu.CompilerParams(use_tc_tiling_on_sc=False,
                                  needs_layout_passes=False)


@functools.partial(
    pl.kernel,
    out_type=jax.ShapeDtypeStruct((NCORE * NPAD,), jnp.float32),
    mesh=_mesh(),
    compiler_params=_SC_PARAMS,
    scratch_types=[
        pltpu.VMEM((BR, ROW), jnp.int32),
        pltpu.VMEM((ROW,), jnp.float32),
        pltpu.VMEM_SHARED((NPAD,), jnp.float32),
        pltpu.SemaphoreType.DMA,
    ],
)
def _deg_kernel(dst_hbm, zeros_hbm, ones_hbm, out_hbm, didx, ones_v, acc, sem):
    c = lax.axis_index("c")
    s = lax.axis_index("s")
    wid = s * NCORE + c
    pltpu.sync_copy(ones_hbm, ones_v)
    pltpu.sync_copy(zeros_hbm, acc.at[pl.ds(s * ZR, ZR)])
    plsc.subcore_barrier()

    def body(j, carry):
        b = wid + j * NW

        @pl.when(b < NBLK)
        def _():
            pltpu.sync_copy(dst_hbm.at[pl.ds(b * BR, BR)], didx)
            sd = [
                pltpu.async_copy(ones_v, acc.at[didx.at[r]], sem, add=True)
                for r in range(BR)
            ]
            for d in sd:
                d.wait()

        return carry

    lax.fori_loop(0, ITERS, body, 0)
    plsc.subcore_barrier()
    pltpu.sync_copy(acc.at[pl.ds(s * ZR, ZR)],
                    out_hbm.at[pl.ds(c * NPAD + s * ZR, ZR)])


def _make_scatter_kernel(feat):
    NSLOT = 3
    JPAD = -(-ITERS // NSLOT) * NSLOT + NSLOT

    @functools.partial(
        pl.kernel,
        out_type=jax.ShapeDtypeStruct((NCORE, feat, NPAD), jnp.float32),
        mesh=_mesh(),
        compiler_params=_SC_PARAMS,
        scratch_types=[
            pltpu.VMEM((NSLOT, BR, ROW), jnp.int32),
            pltpu.VMEM((NSLOT, BR, ROW), jnp.int32),
            pltpu.VMEM((NSLOT, BR * feat, ROW), jnp.float32),
        ] + [
            pltpu.VMEM_SHARED((NPAD,), jnp.float32)
            for _ in range(feat)
        ] + [
            pltpu.SemaphoreType.DMA,
            pltpu.SemaphoreType.DMA,
            pltpu.SemaphoreType.DMA,
        ],
    )
    def _k(src_hbm, dst_hbm, *cols_and_rest):
        cols_hbm = cols_and_rest[:feat]
        zeros_hbm = cols_and_rest[feat]
        out_hbm = cols_and_rest[feat + 1]
        sidx = cols_and_rest[feat + 2]
        didx = cols_and_rest[feat + 3]
        vals = cols_and_rest[feat + 4]
        accs = cols_and_rest[feat + 5:feat + 5 + feat]
        isem = cols_and_rest[feat + 5 + feat]
        gsem = cols_and_rest[feat + 6 + feat]
        ssem = cols_and_rest[feat + 7 + feat]

        c = lax.axis_index("c")
        s = lax.axis_index("s")
        wid = s * NCORE + c
        for k in range(feat):
            pltpu.sync_copy(zeros_hbm, accs[k].at[pl.ds(s * ZR, ZR)])
        plsc.subcore_barrier()

        def idx_issue(blk, slot):
            pltpu.async_copy(src_hbm.at[pl.ds(blk * BR, BR)],
                             sidx.at[slot], isem)
            pltpu.async_copy(dst_hbm.at[pl.ds(blk * BR, BR)],
                             didx.at[slot], isem)

        def idx_drain(blk, slot):
            pltpu.make_async_copy(src_hbm.at[pl.ds(blk * BR, BR)],
                                  sidx.at[slot], isem).wait()
            pltpu.make_async_copy(dst_hbm.at[pl.ds(blk * BR, BR)],
                                  didx.at[slot], isem).wait()

        def gat_issue(slot):
            for r in range(BR):
                for k in range(feat):
                    pltpu.async_copy(cols_hbm[k].at[sidx.at[slot, r]],
                                     vals.at[slot, r * feat + k], gsem)

        def gat_drain(slot):
            for r in range(BR):
                for k in range(feat):
                    pltpu.make_async_copy(
                        cols_hbm[k].at[sidx.at[slot, r]],
                        vals.at[slot, r * feat + k], gsem).wait()

        def sca_issue(slot):
            for r in range(BR):
                for k in range(feat):
                    pltpu.async_copy(vals.at[slot, r * feat + k],
                                     accs[k].at[didx.at[slot, r]], ssem,
                                     add=True)

        def sca_drain(slot):
            for r in range(BR):
                for k in range(feat):
                    pltpu.make_async_copy(
                        vals.at[slot, r * feat + k],
                        accs[k].at[didx.at[slot, r]], ssem).wait()

        def valid(blk):
            return (blk >= 0) & (blk < NBLK * NW)

        b0 = wid

        @pl.when(b0 < NBLK)
        def _():
            idx_issue(b0, 0)

        def outer(j3, carry):
            for pp in range(NSLOT):
                j = j3 * NSLOT + pp
                p = pp
                nxt = (pp + 1) % NSLOT
                prv = (pp + 2) % NSLOT
                b = wid + j * NW

                @pl.when((b - 2 * NW >= 0) & (b - 2 * NW < NBLK))
                def _(slot=nxt):
                    sca_drain(slot)

                @pl.when(b + NW < NBLK)
                def _(blk=b + NW, slot=nxt):
                    idx_issue(blk, slot)

                @pl.when(b < NBLK)
                def _(blk=b, slot=p):
                    idx_drain(blk, slot)
                    gat_issue(slot)

                @pl.when((b - NW >= 0) & (b - NW < NBLK))
                def _(slot=prv):
                    gat_drain(slot)
                    sca_issue(slot)
            return carry

        lax.fori_loop(0, JPAD // NSLOT, outer, 0)
        plsc.subcore_barrier()
        for k in range(feat):
            pltpu.sync_copy(accs[k].at[pl.ds(s * ZR, ZR)],
                            out_hbm.at[c, k, pl.ds(s * ZR, ZR)])

    return _k


_scatter4 = _make_scatter_kernel(4)
_scatter2 = _make_scatter_kernel(2)


def _tc1_body(degp_ref, xc_ref, w1_ref, dis_ref, g1_ref):
    deg = degp_ref[0] + degp_ref[1] + 1.0
    dis = lax.rsqrt(deg)
    dis_ref[...] = dis
    xs = [xc_ref[k] for k in range(3)]
    for j in range(4):
        h = xs[0] * w1_ref[0, j] + xs[1] * w1_ref[1, j] + xs[2] * w1_ref[2, j]
        g1_ref[j] = h * dis


def _tc2_body(t1p_ref, g1_ref, dis_ref, b1_ref, w2_ref, g2_ref):
    dis = dis_ref[...]
    hs = []
    for k in range(4):
        t = (t1p_ref[0, k] + t1p_ref[1, k] + g1_ref[k]) * dis + b1_ref[k]
        hs.append(jnp.maximum(t, 0.0))
    for j in range(2):
        h = (hs[0] * w2_ref[0, j] + hs[1] * w2_ref[1, j]
             + hs[2] * w2_ref[2, j] + hs[3] * w2_ref[3, j])
        g2_ref[j] = h * dis


def _tc3_body(t2p_ref, g2_ref, dis_ref, b2_ref, out_ref):
    dis = dis_ref[...]
    h0 = (t2p_ref[0, 0] + t2p_ref[1, 0] + g2_ref[0]) * dis + b2_ref[0]
    h1 = (t2p_ref[0, 1] + t2p_ref[1, 1] + g2_ref[1]) * dis + b2_ref[1]
    m = jnp.maximum(h0, h1)
    lse = m + jnp.log(jnp.exp(h0 - m) + jnp.exp(h1 - m))
    out_ref[0] = h0 - lse
    out_ref[1] = h1 - lse


def kernel(x, edge_index, W1, b1, W2, b2):
    f32 = jnp.float32
    ei = edge_index.astype(jnp.int32)
    src = ei[0].reshape(NROWS, ROW)
    dst = ei[1].reshape(NROWS, ROW)

    xcols = jnp.pad(x, ((0, NPAD - N_NODES), (0, 0))).T.reshape(3, LANES, 128)

    zeros1 = jnp.zeros((ZR,), f32)
    ones = jnp.ones((ROW,), f32)

    vmem = pl.BlockSpec(memory_space=pltpu.VMEM)
    smem = pl.BlockSpec(memory_space=pltpu.SMEM)

    degp = _deg_kernel(dst, zeros1, ones).reshape(NCORE, LANES, 128)

    dis, g1 = pl.pallas_call(
        _tc1_body,
        out_shape=[
            jax.ShapeDtypeStruct((LANES, 128), f32),
            jax.ShapeDtypeStruct((4, LANES, 128), f32),
        ],
        in_specs=[vmem, vmem, smem],
        out_specs=[vmem, vmem],
    )(degp, xcols, W1)

    g1f = g1.reshape(4, NPAD)
    t1p = _scatter4(src, dst, g1f[0], g1f[1], g1f[2], g1f[3], zeros1)

    g2 = pl.pallas_call(
        _tc2_body,
        out_shape=jax.ShapeDtypeStruct((2, LANES, 128), f32),
        in_specs=[vmem, vmem, vmem, smem, smem],
        out_specs=vmem,
    )(t1p.reshape(NCORE, 4, LANES, 128), g1, dis, b1, W2)

    g2f = g2.reshape(2, NPAD)
    t2p = _scatter2(src, dst, g2f[0], g2f[1], zeros1)

    outc = pl.pallas_call(
        _tc3_body,
        out_shape=jax.ShapeDtypeStruct((2, LANES, 128), f32),
        in_specs=[vmem, vmem, vmem, smem],
        out_specs=vmem,
    )(t2p.reshape(NCORE, 2, LANES, 128), g2, dis, b2)

    return outc.reshape(2, NPAD)[:, :N_NODES].T


# speedup vs baseline: 83.4423x; 1.0000x over previous
import functools

import jax
import jax.numpy as jnp
from jax import lax
from jax.experimental import pallas as pl
from jax.experimental.pallas import tpu as pltpu
from jax.experimental.pallas import tpu_sc as plsc

N_NODES = 100000
N_EDGES = 6400000

ROW = 512
BR = 2
NROWS = N_EDGES // ROW
NBLK = NROWS // BR
NCORE = 2
NSUB = 16
NW = NCORE * NSUB
ITERS = -(-NBLK // NW)
NPAD = 100352
ZR = NPAD // NSUB
LANES = NPAD // 128


def _mesh():
    return plsc.VectorSubcoreMesh(core_axis_name="c", subcore_axis_name="s")


_SC_PARAMS = pltpu.CompilerParams(use_tc_tiling_on_sc=False,
                                  needs_layout_passes=False)


@functools.partial(
    pl.kernel,
    out_type=jax.ShapeDtypeStruct((NCORE * NPAD,), jnp.float32),
    mesh=_mesh(),
    compiler_params=_SC_PARAMS,
    scratch_types=[
        pltpu.VMEM((BR, ROW), jnp.int32),
        pltpu.VMEM((ROW,), jnp.float32),
        pltpu.VMEM_SHARED((NPAD,), jnp.float32),
        pltpu.SemaphoreType.DMA,
    ],
)
def _deg_kernel(dst_hbm, zeros_hbm, ones_hbm, out_hbm, didx, ones_v, acc, sem):
    c = lax.axis_index("c")
    s = lax.axis_index("s")
    wid = s * NCORE + c
    pltpu.sync_copy(ones_hbm, ones_v)
    pltpu.sync_copy(zeros_hbm, acc.at[pl.ds(s * ZR, ZR)])
    plsc.subcore_barrier()

    def body(j, carry):
        b = wid + j * NW

        @pl.when(b < NBLK)
        def _():
            pltpu.sync_copy(dst_hbm.at[pl.ds(b * BR, BR)], didx)
            sd = [
                pltpu.async_copy(ones_v, acc.at[didx.at[r]], sem, add=True)
                for r in range(BR)
            ]
            for d in sd:
                d.wait()

        return carry

    lax.fori_loop(0, ITERS, body, 0)
    plsc.subcore_barrier()
    pltpu.sync_copy(acc.at[pl.ds(s * ZR, ZR)],
                    out_hbm.at[pl.ds(c * NPAD + s * ZR, ZR)])


def _make_scatter_kernel(feat):
    NSLOT = 3
    JPAD = -(-ITERS // NSLOT) * NSLOT + NSLOT

    @functools.partial(
        pl.kernel,
        out_type=jax.ShapeDtypeStruct((NCORE, feat, NPAD), jnp.float32),
        mesh=_mesh(),
        compiler_params=_SC_PARAMS,
        scratch_types=[
            pltpu.VMEM((NSLOT, BR, ROW), jnp.int32),
            pltpu.VMEM((NSLOT, BR, ROW), jnp.int32),
            pltpu.VMEM((NSLOT, BR * feat, ROW), jnp.float32),
        ] + [
            pltpu.VMEM_SHARED((NPAD,), jnp.float32)
            for _ in range(feat)
        ] + [
            pltpu.SemaphoreType.DMA,
            pltpu.SemaphoreType.DMA,
            pltpu.SemaphoreType.DMA,
        ],
    )
    def _k(src_hbm, dst_hbm, *cols_and_rest):
        cols_hbm = cols_and_rest[:feat]
        zeros_hbm = cols_and_rest[feat]
        out_hbm = cols_and_rest[feat + 1]
        sidx = cols_and_rest[feat + 2]
        didx = cols_and_rest[feat + 3]
        vals = cols_and_rest[feat + 4]
        accs = cols_and_rest[feat + 5:feat + 5 + feat]
        isem = cols_and_rest[feat + 5 + feat]
        gsem = cols_and_rest[feat + 6 + feat]
        ssem = cols_and_rest[feat + 7 + feat]

        c = lax.axis_index("c")
        s = lax.axis_index("s")
        wid = s * NCORE + c
        for k in range(feat):
            pltpu.sync_copy(zeros_hbm, accs[k].at[pl.ds(s * ZR, ZR)])
        plsc.subcore_barrier()

        def idx_issue(blk, slot):
            pltpu.async_copy(src_hbm.at[pl.ds(blk * BR, BR)],
                             sidx.at[slot], isem)
            pltpu.async_copy(dst_hbm.at[pl.ds(blk * BR, BR)],
                             didx.at[slot], isem)

        def idx_drain(blk, slot):
            pltpu.make_async_copy(src_hbm.at[pl.ds(blk * BR, BR)],
                                  sidx.at[slot], isem).wait()
            pltpu.make_async_copy(dst_hbm.at[pl.ds(blk * BR, BR)],
                                  didx.at[slot], isem).wait()

        def gat_issue(slot):
            for r in range(BR):
                for k in range(feat):
                    pltpu.async_copy(cols_hbm[k].at[sidx.at[slot, r]],
                                     vals.at[slot, r * feat + k], gsem)

        def gat_drain(slot):
            for r in range(BR):
                for k in range(feat):
                    pltpu.make_async_copy(
                        cols_hbm[k].at[sidx.at[slot, r]],
                        vals.at[slot, r * feat + k], gsem).wait()

        def sca_issue(slot):
            for r in range(BR):
                for k in range(feat):
                    pltpu.async_copy(vals.at[slot, r * feat + k],
                                     accs[k].at[didx.at[slot, r]], ssem,
                                     add=True)

        def sca_drain(slot):
            for r in range(BR):
                for k in range(feat):
                    pltpu.make_async_copy(
                        vals.at[slot, r * feat + k],
                        accs[k].at[didx.at[slot, r]], ssem).wait()

        def valid(blk):
            return (blk >= 0) & (blk < NBLK * NW)

        b0 = wid

        @pl.when(b0 < NBLK)
        def _():
            idx_issue(b0, 0)

        def outer(j3, carry):
            for pp in range(NSLOT):
                j = j3 * NSLOT + pp
                p = pp
                nxt = (pp + 1) % NSLOT
                prv = (pp + 2) % NSLOT
                b = wid + j * NW

                @pl.when((b - 2 * NW >= 0) & (b - 2 * NW < NBLK))
                def _(slot=nxt):
                    sca_drain(slot)

                @pl.when(b + NW < NBLK)
                def _(blk=b + NW, slot=nxt):
                    idx_issue(blk, slot)

                @pl.when(b < NBLK)
                def _(blk=b, slot=p):
                    idx_drain(blk, slot)
                    gat_issue(slot)

                @pl.when((b - NW >= 0) & (b - NW < NBLK))
                def _(slot=prv):
                    gat_drain(slot)
                    sca_issue(slot)
            return carry

        lax.fori_loop(0, JPAD // NSLOT, outer, 0)
        plsc.subcore_barrier()
        for k in range(feat):
            pltpu.sync_copy(accs[k].at[pl.ds(s * ZR, ZR)],
                            out_hbm.at[c, k, pl.ds(s * ZR, ZR)])

    return _k


_scatter4 = _make_scatter_kernel(4)
_scatter2 = _make_scatter_kernel(2)


def _tc1_body(degp_ref, xc_ref, w1_ref, dis_ref, g1_ref):
    deg = degp_ref[0] + degp_ref[1] + 1.0
    dis = lax.rsqrt(deg)
    dis_ref[...] = dis
    xs = [xc_ref[k] for k in range(3)]
    for j in range(4):
        h = xs[0] * w1_ref[0, j] + xs[1] * w1_ref[1, j] + xs[2] * w1_ref[2, j]
        g1_ref[j] = h * dis


def _tc2_body(t1p_ref, g1_ref, dis_ref, b1_ref, w2_ref, g2_ref):
    dis = dis_ref[...]
    hs = []
    for k in range(4):
        t = (t1p_ref[0, k] + t1p_ref[1, k] + g1_ref[k]) * dis + b1_ref[k]
        hs.append(jnp.maximum(t, 0.0))
    for j in range(2):
        h = (hs[0] * w2_ref[0, j] + hs[1] * w2_ref[1, j]
             + hs[2] * w2_ref[2, j] + hs[3] * w2_ref[3, j])
        g2_ref[j] = h * dis


def _tc3_body(t2p_ref, g2_ref, dis_ref, b2_ref, out_ref):
    dis = dis_ref[...]
    h0 = (t2p_ref[0, 0] + t2p_ref[1, 0] + g2_ref[0]) * dis + b2_ref[0]
    h1 = (t2p_ref[0, 1] + t2p_ref[1, 1] + g2_ref[1]) * dis + b2_ref[1]
    m = jnp.maximum(h0, h1)
    lse = m + jnp.log(jnp.exp(h0 - m) + jnp.exp(h1 - m))
    out_ref[0] = h0 - lse
    out_ref[1] = h1 - lse


def kernel(x, edge_index, W1, b1, W2, b2):
    f32 = jnp.float32
    ei = edge_index.astype(jnp.int32)
    src = ei[0].reshape(NROWS, ROW)
    dst = ei[1].reshape(NROWS, ROW)

    xcols = jnp.pad(x, ((0, NPAD - N_NODES), (0, 0))).T.reshape(3, LANES, 128)

    zeros1 = jnp.zeros((ZR,), f32)
    ones = jnp.ones((ROW,), f32)

    vmem = pl.BlockSpec(memory_space=pltpu.VMEM)
    smem = pl.BlockSpec(memory_space=pltpu.SMEM)

    degp = _deg_kernel(dst, zeros1, ones).reshape(NCORE, LANES, 128)

    dis, g1 = pl.pallas_call(
        _tc1_body,
        out_shape=[
            jax.ShapeDtypeStruct((LANES, 128), f32),
            jax.ShapeDtypeStruct((4, LANES, 128), f32),
        ],
        in_specs=[vmem, vmem, smem],
        out_specs=[vmem, vmem],
    )(degp, xcols, W1)

    g1f = g1.reshape(4, NPAD)
    t1p = _scatter4(src, dst, g1f[0], g1f[1], g1f[2], g1f[3], zeros1)

    g2 = pl.pallas_call(
        _tc2_body,
        out_shape=jax.ShapeDtypeStruct((2, LANES, 128), f32),
        in_specs=[vmem, vmem, vmem, smem, smem],
        out_specs=vmem,
    )(t1p.reshape(NCORE, 4, LANES, 128), g1, dis, b1, W2)

    g2f = g2.reshape(2, NPAD)
    t2p = _scatter2(src, dst, g2f[0], g2f[1], zeros1)

    outc = pl.pallas_call(
        _tc3_body,
        out_shape=jax.ShapeDtypeStruct((2, LANES, 128), f32),
        in_specs=[vmem, vmem, vmem, smem],
        out_specs=vmem,
    )(t2p.reshape(NCORE, 2, LANES, 128), g2, dis, b2)

    return outc.reshape(2, NPAD)[:, :N_NODES].T


# speedup vs baseline: 83.4766x; 1.0004x over previous
import functools

import jax
import jax.numpy as jnp
from jax import lax
from jax.experimental import pallas as pl
from jax.experimental.pallas import tpu as pltpu
from jax.experimental.pallas import tpu_sc as plsc

N_NODES = 100000
N_EDGES = 6400000

ROW = 512
BR = 2
NROWS = N_EDGES // ROW
NBLK = NROWS // BR
NCORE = 2
NSUB = 16
NW = NCORE * NSUB
ITERS = -(-NBLK // NW)
NPAD = 100352
ZR = NPAD // NSUB
LANES = NPAD // 128


def _mesh():
    return plsc.VectorSubcoreMesh(core_axis_name="c", subcore_axis_name="s")


_SC_PARAMS = pltpu.CompilerParams(use_tc_tiling_on_sc=False,
                                  needs_layout_passes=False)


@functools.partial(
    pl.kernel,
    out_type=jax.ShapeDtypeStruct((NCORE * NPAD,), jnp.float32),
    mesh=_mesh(),
    compiler_params=_SC_PARAMS,
    scratch_types=[
        pltpu.VMEM((BR, ROW), jnp.int32),
        pltpu.VMEM((ROW,), jnp.float32),
        pltpu.VMEM_SHARED((NPAD,), jnp.float32),
        pltpu.SemaphoreType.DMA,
    ],
)
def _deg_kernel(dst_hbm, zeros_hbm, ones_hbm, out_hbm, didx, ones_v, acc, sem):
    c = lax.axis_index("c")
    s = lax.axis_index("s")
    wid = s * NCORE + c
    pltpu.sync_copy(ones_hbm, ones_v)
    pltpu.sync_copy(zeros_hbm, acc.at[pl.ds(s * ZR, ZR)])
    plsc.subcore_barrier()

    def body(j, carry):
        b = wid + j * NW

        @pl.when(b < NBLK)
        def _():
            pltpu.sync_copy(dst_hbm.at[pl.ds(b * BR, BR)], didx)
            sd = [
                pltpu.async_copy(ones_v, acc.at[didx.at[r]], sem, add=True)
                for r in range(BR)
            ]
            for d in sd:
                d.wait()

        return carry

    lax.fori_loop(0, ITERS, body, 0)
    plsc.subcore_barrier()
    pltpu.sync_copy(acc.at[pl.ds(s * ZR, ZR)],
                    out_hbm.at[pl.ds(c * NPAD + s * ZR, ZR)])


def _make_scatter_kernel(feat):
    NSLOT = 3
    JPAD = -(-ITERS // NSLOT) * NSLOT + NSLOT

    @functools.partial(
        pl.kernel,
        out_type=jax.ShapeDtypeStruct((NCORE, feat, NPAD), jnp.float32),
        mesh=_mesh(),
        compiler_params=_SC_PARAMS,
        scratch_types=[
            pltpu.VMEM((NSLOT, BR, ROW), jnp.int32),
            pltpu.VMEM((NSLOT, BR, ROW), jnp.int32),
            pltpu.VMEM((NSLOT, BR * feat, ROW), jnp.float32),
        ] + [
            pltpu.VMEM_SHARED((NPAD,), jnp.float32)
            for _ in range(feat)
        ] + [
            pltpu.SemaphoreType.DMA,
            pltpu.SemaphoreType.DMA,
            pltpu.SemaphoreType.DMA,
        ],
    )
    def _k(src_hbm, dst_hbm, *cols_and_rest):
        cols_hbm = cols_and_rest[:feat]
        zeros_hbm = cols_and_rest[feat]
        out_hbm = cols_and_rest[feat + 1]
        sidx = cols_and_rest[feat + 2]
        didx = cols_and_rest[feat + 3]
        vals = cols_and_rest[feat + 4]
        accs = cols_and_rest[feat + 5:feat + 5 + feat]
        isem = cols_and_rest[feat + 5 + feat]
        gsem = cols_and_rest[feat + 6 + feat]
        ssem = cols_and_rest[feat + 7 + feat]

        c = lax.axis_index("c")
        s = lax.axis_index("s")
        wid = s * NCORE + c
        for k in range(feat):
            pltpu.sync_copy(zeros_hbm, accs[k].at[pl.ds(s * ZR, ZR)])
        plsc.subcore_barrier()

        def idx_issue(blk, slot):
            pltpu.async_copy(src_hbm.at[pl.ds(blk * BR, BR)],
                             sidx.at[slot], isem)
            pltpu.async_copy(dst_hbm.at[pl.ds(blk * BR, BR)],
                             didx.at[slot], isem)

        def idx_drain(blk, slot):
            pltpu.make_async_copy(src_hbm.at[pl.ds(blk * BR, BR)],
                                  sidx.at[slot], isem).wait()
            pltpu.make_async_copy(dst_hbm.at[pl.ds(blk * BR, BR)],
                                  didx.at[slot], isem).wait()

        def gat_issue(slot):
            for r in range(BR):
                for k in range(feat):
                    pltpu.async_copy(cols_hbm[k].at[sidx.at[slot, r]],
                                     vals.at[slot, r * feat + k], gsem)

        def gat_drain(slot):
            for r in range(BR):
                for k in range(feat):
                    pltpu.make_async_copy(
                        cols_hbm[k].at[sidx.at[slot, r]],
                        vals.at[slot, r * feat + k], gsem).wait()

        def sca_issue(slot):
            for r in range(BR):
                for k in range(feat):
                    pltpu.async_copy(vals.at[slot, r * feat + k],
                                     accs[k].at[didx.at[slot, r]], ssem,
                                     add=True)

        def sca_drain(slot):
            for r in range(BR):
                for k in range(feat):
                    pltpu.make_async_copy(
                        vals.at[slot, r * feat + k],
                        accs[k].at[didx.at[slot, r]], ssem).wait()

        b0 = wid

        @pl.when(b0 < NBLK)
        def _():
            idx_issue(b0, 0)

        def outer(j3, carry):
            for pp in range(NSLOT):
                j = j3 * NSLOT + pp
                p = pp
                nxt = (pp + 1) % NSLOT
                prv = (pp + 2) % NSLOT
                b = wid + j * NW

                @pl.when((b - 2 * NW >= 0) & (b - 2 * NW < NBLK))
                def _(slot=nxt):
                    sca_drain(slot)

                @pl.when(b + NW < NBLK)
                def _(blk=b + NW, slot=nxt):
                    idx_issue(blk, slot)

                @pl.when(b < NBLK)
                def _(blk=b, slot=p):
                    idx_drain(blk, slot)
                    gat_issue(slot)

                @pl.when((b - NW >= 0) & (b - NW < NBLK))
                def _(slot=prv):
                    gat_drain(slot)
                    sca_issue(slot)
            return carry

        lax.fori_loop(0, JPAD // NSLOT, outer, 0)
        plsc.subcore_barrier()
        for k in range(feat):
            pltpu.sync_copy(accs[k].at[pl.ds(s * ZR, ZR)],
                            out_hbm.at[c, k, pl.ds(s * ZR, ZR)])

    return _k


_scatter4 = _make_scatter_kernel(4)
_scatter2 = _make_scatter_kernel(2)


def _tc1_body(degp_ref, xc_ref, w1_ref, dis_ref, g1_ref):
    deg = degp_ref[0] + degp_ref[1] + 1.0
    dis = lax.rsqrt(deg)
    dis_ref[...] = dis
    xs = [xc_ref[k] for k in range(3)]
    for j in range(4):
        h = xs[0] * w1_ref[0, j] + xs[1] * w1_ref[1, j] + xs[2] * w1_ref[2, j]
        g1_ref[j] = h * dis


def _tc2_body(t1p_ref, g1_ref, dis_ref, b1_ref, w2_ref, g2_ref):
    dis = dis_ref[...]
    hs = []
    for k in range(4):
        t = (t1p_ref[0, k] + t1p_ref[1, k] + g1_ref[k]) * dis + b1_ref[k]
        hs.append(jnp.maximum(t, 0.0))
    for j in range(2):
        h = (hs[0] * w2_ref[0, j] + hs[1] * w2_ref[1, j]
             + hs[2] * w2_ref[2, j] + hs[3] * w2_ref[3, j])
        g2_ref[j] = h * dis


def _tc3_body(t2p_ref, g2_ref, dis_ref, b2_ref, out_ref):
    dis = dis_ref[...]
    h0 = (t2p_ref[0, 0] + t2p_ref[1, 0] + g2_ref[0]) * dis + b2_ref[0]
    h1 = (t2p_ref[0, 1] + t2p_ref[1, 1] + g2_ref[1]) * dis + b2_ref[1]
    m = jnp.maximum(h0, h1)
    lse = m + jnp.log(jnp.exp(h0 - m) + jnp.exp(h1 - m))
    out_ref[0] = h0 - lse
    out_ref[1] = h1 - lse


def kernel(x, edge_index, W1, b1, W2, b2):
    f32 = jnp.float32
    ei = edge_index.astype(jnp.int32)
    src = ei[0].reshape(NROWS, ROW)
    dst = ei[1].reshape(NROWS, ROW)

    xcols = jnp.pad(x, ((0, NPAD - N_NODES), (0, 0))).T.reshape(3, LANES, 128)

    zeros1 = jnp.zeros((ZR,), f32)
    ones = jnp.ones((ROW,), f32)

    vmem = pl.BlockSpec(memory_space=pltpu.VMEM)
    smem = pl.BlockSpec(memory_space=pltpu.SMEM)

    degp = _deg_kernel(dst, zeros1, ones).reshape(NCORE, LANES, 128)

    dis, g1 = pl.pallas_call(
        _tc1_body,
        out_shape=[
            jax.ShapeDtypeStruct((LANES, 128), f32),
            jax.ShapeDtypeStruct((4, LANES, 128), f32),
        ],
        in_specs=[vmem, vmem, smem],
        out_specs=[vmem, vmem],
    )(degp, xcols, W1)

    g1f = g1.reshape(4, NPAD)
    t1p = _scatter4(src, dst, g1f[0], g1f[1], g1f[2], g1f[3], zeros1)

    g2 = pl.pallas_call(
        _tc2_body,
        out_shape=jax.ShapeDtypeStruct((2, LANES, 128), f32),
        in_specs=[vmem, vmem, vmem, smem, smem],
        out_specs=vmem,
    )(t1p.reshape(NCORE, 4, LANES, 128), g1, dis, b1, W2)

    g2f = g2.reshape(2, NPAD)
    t2p = _scatter2(src, dst, g2f[0], g2f[1], zeros1)

    outc = pl.pallas_call(
        _tc3_body,
        out_shape=jax.ShapeDtypeStruct((2, LANES, 128), f32),
        in_specs=[vmem, vmem, vmem, smem],
        out_specs=vmem,
    )(t2p.reshape(NCORE, 2, LANES, 128), g2, dis, b2)

    return outc.reshape(2, NPAD)[:, :N_NODES].T
